# Initial kernel scaffold; baseline (speedup 1.0000x reference)
import jax
import jax.numpy as jnp
from jax.experimental import pallas as pl


def kernel(x, emb_weight):
    raise NotImplementedError("write your pallas kernel here")


# speedup vs baseline: 1.9668x; 1.9668x over previous
import jax
import jax.numpy as jnp
from jax.experimental import pallas as pl


def _add_kernel(x_ref, emb_ref, o_ref):
    o_ref[...] = x_ref[...] + emb_ref[...][None, :, :]


def kernel(x, emb_weight):
    batch, seq_len, d_model = x.shape

    s_blk = 512
    while seq_len % s_blk:
        s_blk //= 2
    num_s = seq_len // s_blk

    return pl.pallas_call(
        _add_kernel,
        grid=(num_s,),
        in_specs=[
            pl.BlockSpec((batch, s_blk, d_model), lambda s: (0, s, 0)),
            pl.BlockSpec((s_blk, d_model), lambda s: (s, 0)),
        ],
        out_specs=pl.BlockSpec((batch, s_blk, d_model), lambda s: (0, s, 0)),
        out_shape=jax.ShapeDtypeStruct((batch, seq_len, d_model), x.dtype),
    )(x, emb_weight)
